# baseline (speedup 1.0000x reference)
import functools
import math

import jax
import jax.numpy as jnp
from jax import lax
from jax.experimental import pallas as pl
from jax.experimental.pallas import tpu as pltpu
from jax.experimental.pallas import tpu_sc as plsc

HIDDEN = 64
SCALE = math.sqrt(HIDDEN)

NC = 2
NS = 16
NW = NC * NS

B = 16384 * 50
BPW = B // NW
SUB = 128
CHUNK = 512
NSUB = CHUNK // SUB
NCHUNK = BPW // CHUNK
IDX_ROWS = BPW // SUB

_mesh = plsc.VectorSubcoreMesh(core_axis_name="c", subcore_axis_name="s")


@functools.partial(
    pl.kernel,
    mesh=_mesh,
    out_type=jax.ShapeDtypeStruct((B, HIDDEN), jnp.float32),
    scratch_types=[
        pltpu.VMEM((IDX_ROWS, SUB), jnp.int32),
        pltpu.VMEM((CHUNK, HIDDEN), jnp.float32),
        pltpu.SemaphoreType.DMA,
    ],
    compiler_params=pltpu.CompilerParams(use_tc_tiling_on_sc=False),
)
def _emb_lookup(x_hbm, table_hbm, out_hbm, idx_v, rows_v, sem):
    wid = lax.axis_index("s") * NC + lax.axis_index("c")
    pltpu.sync_copy(x_hbm.at[pl.ds(wid * IDX_ROWS, IDX_ROWS)], idx_v)

    def chunk_body(c, _):
        copies = []
        for s in range(NSUB):
            copies.append(
                pltpu.async_copy(
                    table_hbm.at[idx_v.at[c * NSUB + s]],
                    rows_v.at[pl.ds(s * SUB, SUB)],
                    sem,
                )
            )
        for cp in copies:
            cp.wait()

        def row_body(r, _):
            for g in range(HIDDEN // 16):
                sl = pl.ds(g * 16, 16)
                rows_v[r, sl] = rows_v[r, sl] * SCALE
            return 0

        lax.fori_loop(0, CHUNK, row_body, 0)

        pltpu.sync_copy(
            rows_v, out_hbm.at[pl.ds(wid * BPW + c * CHUNK, CHUNK)]
        )
        return 0

    lax.fori_loop(0, NCHUNK, chunk_body, 0)


def kernel(x, table):
    n_tok, seq = x.shape
    assert n_tok * seq == B and table.shape[1] == HIDDEN
    x2 = x.astype(jnp.int32).reshape(B // SUB, SUB)
    out = _emb_lookup(x2, table)
    return out.reshape(n_tok, seq, HIDDEN)
